# baseline (speedup 1.0000x reference)
import functools

import jax
import jax.numpy as jnp
from jax import lax
from jax.experimental import pallas as pl
from jax.experimental.pallas import tpu as pltpu
from jax.experimental.pallas import tpu_sc as plsc

VOCAB_SIZE = 100000
EMBED_D = 32
BATCH = 1024
LN_EPS = 1e-5
TILE_V = 2048

_NUM_CORES = 2
_NUM_SUBCORES = 16
_NUM_WORKERS = _NUM_CORES * _NUM_SUBCORES
_B_PER_W = BATCH // _NUM_WORKERS


def _sc_gather_body(table_hbm, idx_hbm, out_hbm, idx_v, rows_v, sem):
    wid = lax.axis_index("s") * _NUM_CORES + lax.axis_index("c")
    base = wid * _B_PER_W
    pltpu.sync_copy(idx_hbm.at[pl.ds(base, _B_PER_W)], idx_v)
    pltpu.async_copy(table_hbm.at[idx_v], rows_v, sem).wait()
    pltpu.sync_copy(rows_v, out_hbm.at[pl.ds(base, _B_PER_W)])


def _sc_gather(table, idx):
    mesh = plsc.VectorSubcoreMesh(core_axis_name="c", subcore_axis_name="s")
    fn = pl.kernel(
        _sc_gather_body,
        mesh=mesh,
        out_type=jax.ShapeDtypeStruct((BATCH, EMBED_D), jnp.float32),
        scratch_types=[
            pltpu.VMEM((_B_PER_W,), jnp.int32),
            pltpu.VMEM((_B_PER_W, EMBED_D), jnp.float32),
            pltpu.SemaphoreType.DMA,
        ],
        compiler_params=pltpu.CompilerParams(use_tc_tiling_on_sc=False),
    )
    return fn(table, idx)


def _head_body(h0_ref, w1_ref, gamma_ref, beta_ref, wh_ref, bh_ref, out_ref,
               h_ref):
    @pl.when(pl.program_id(0) == 0)
    def _():
        h = lax.dot_general(h0_ref[...].astype(jnp.bfloat16),
                            w1_ref[...].astype(jnp.bfloat16),
                            (((1,), (1,)), ((), ())),
                            preferred_element_type=jnp.float32)
        h = jnp.maximum(h, 0.0)
        mean = jnp.mean(h, axis=-1, keepdims=True)
        cen = h - mean
        var = jnp.mean(cen * cen, axis=-1, keepdims=True)
        hn = cen * lax.rsqrt(var + LN_EPS)
        h_ref[...] = (hn * gamma_ref[...] + beta_ref[...]).astype(jnp.bfloat16)

    out_ref[...] = lax.dot_general(
        h_ref[...], wh_ref[...].astype(jnp.bfloat16),
        (((1,), (1,)), ((), ())),
        preferred_element_type=jnp.float32) + bh_ref[...]


def _head(h0, W1, gamma2, beta2, Wh, bh2):
    grid = (pl.cdiv(VOCAB_SIZE, TILE_V),)
    return pl.pallas_call(
        _head_body,
        grid=grid,
        in_specs=[
            pl.BlockSpec((BATCH, EMBED_D), lambda i: (0, 0)),
            pl.BlockSpec((EMBED_D, EMBED_D), lambda i: (0, 0)),
            pl.BlockSpec((1, EMBED_D), lambda i: (0, 0)),
            pl.BlockSpec((1, EMBED_D), lambda i: (0, 0)),
            pl.BlockSpec((TILE_V, EMBED_D), lambda i: (i, 0)),
            pl.BlockSpec((1, TILE_V), lambda i: (0, i)),
        ],
        out_specs=pl.BlockSpec((BATCH, TILE_V), lambda i: (0, i)),
        out_shape=jax.ShapeDtypeStruct((BATCH, VOCAB_SIZE), jnp.float32),
        scratch_shapes=[pltpu.VMEM((BATCH, EMBED_D), jnp.bfloat16)],
    )(h0, W1, gamma2, beta2, Wh, bh2)


def kernel(x, table, W1, gamma, beta, Wh, bh):
    h0 = _sc_gather(table, x.astype(jnp.int32))
    return _head(h0, W1, gamma.reshape(1, EMBED_D), beta.reshape(1, EMBED_D),
                 Wh, bh.reshape(1, VOCAB_SIZE))


# speedup vs baseline: 2.9610x; 2.9610x over previous
import jax
import jax.numpy as jnp
from jax import lax
from jax.experimental import pallas as pl
from jax.experimental.pallas import tpu as pltpu
from jax.experimental.pallas import tpu_sc as plsc

VOCAB_SIZE = 100000
EMBED_D = 32
BATCH = 1024
LN_EPS = 1e-5
TILE_V = 2048

_NUM_CORES = 2
_NUM_SUBCORES = 16
_NUM_WORKERS = _NUM_CORES * _NUM_SUBCORES
_B_PER_W = BATCH // _NUM_WORKERS


def _sc_gather_body(table_hbm, idx_hbm, out_hbm, idx_v, rows_v, sem):
    wid = lax.axis_index("s") * _NUM_CORES + lax.axis_index("c")
    base = wid * _B_PER_W
    pltpu.sync_copy(idx_hbm.at[pl.ds(base, _B_PER_W)], idx_v)
    pltpu.async_copy(table_hbm.at[idx_v], rows_v, sem).wait()
    pltpu.sync_copy(rows_v, out_hbm.at[pl.ds(base, _B_PER_W)])


def _sc_gather(table, idx):
    mesh = plsc.VectorSubcoreMesh(core_axis_name="c", subcore_axis_name="s")
    fn = pl.kernel(
        _sc_gather_body,
        mesh=mesh,
        out_type=jax.ShapeDtypeStruct((BATCH, EMBED_D), jnp.float32),
        scratch_types=[
            pltpu.VMEM((_B_PER_W,), jnp.int32),
            pltpu.VMEM((_B_PER_W, EMBED_D), jnp.float32),
            pltpu.SemaphoreType.DMA,
        ],
        compiler_params=pltpu.CompilerParams(use_tc_tiling_on_sc=False),
    )
    return fn(table, idx)


def _head_body(h0_ref, w1_ref, gammac_ref, betac_ref, whT_ref, bh_ref,
               out_ref, hT_ref):
    @pl.when(pl.program_id(0) == 0)
    def _():
        hT = lax.dot_general(w1_ref[...].astype(jnp.bfloat16),
                             h0_ref[...].astype(jnp.bfloat16),
                             (((1,), (1,)), ((), ())),
                             preferred_element_type=jnp.float32)
        hT = jnp.maximum(hT, 0.0)
        mean = jnp.mean(hT, axis=0, keepdims=True)
        cen = hT - mean
        var = jnp.mean(cen * cen, axis=0, keepdims=True)
        hTn = cen * lax.rsqrt(var + LN_EPS)
        hT_ref[...] = (hTn * gammac_ref[...] + betac_ref[...]).astype(
            jnp.bfloat16)

    acc = lax.dot_general(whT_ref[...].astype(jnp.bfloat16), hT_ref[...],
                          (((0,), (0,)), ((), ())),
                          preferred_element_type=jnp.float32)
    ones = jnp.ones((1, BATCH), dtype=jnp.bfloat16)
    bias = lax.dot_general(bh_ref[...].astype(jnp.bfloat16), ones,
                           (((0,), (0,)), ((), ())),
                           preferred_element_type=jnp.float32)
    out_ref[...] = acc + bias


def _head(h0, W1, gammac, betac, whT, bh2):
    grid = (pl.cdiv(VOCAB_SIZE, TILE_V),)
    return pl.pallas_call(
        _head_body,
        grid=grid,
        in_specs=[
            pl.BlockSpec((BATCH, EMBED_D), lambda i: (0, 0)),
            pl.BlockSpec((EMBED_D, EMBED_D), lambda i: (0, 0)),
            pl.BlockSpec((EMBED_D, 1), lambda i: (0, 0)),
            pl.BlockSpec((EMBED_D, 1), lambda i: (0, 0)),
            pl.BlockSpec((EMBED_D, TILE_V), lambda i: (0, i)),
            pl.BlockSpec((1, TILE_V), lambda i: (0, i)),
        ],
        out_specs=pl.BlockSpec((TILE_V, BATCH), lambda i: (i, 0)),
        out_shape=jax.ShapeDtypeStruct((VOCAB_SIZE, BATCH), jnp.float32),
        scratch_shapes=[pltpu.VMEM((EMBED_D, BATCH), jnp.bfloat16)],
    )(h0, W1, gammac, betac, whT, bh2)


def kernel(x, table, W1, gamma, beta, Wh, bh):
    h0 = _sc_gather(table, x.astype(jnp.int32))
    lt = _head(h0, W1, gamma.reshape(EMBED_D, 1), beta.reshape(EMBED_D, 1),
               Wh.T, bh.reshape(1, VOCAB_SIZE))
    return lt.T


# speedup vs baseline: 3.5374x; 1.1947x over previous
import jax
import jax.numpy as jnp
from jax import lax
from jax.experimental import pallas as pl
from jax.experimental.pallas import tpu as pltpu
from jax.experimental.pallas import tpu_sc as plsc

VOCAB_SIZE = 100000
EMBED_D = 32
BATCH = 1024
LN_EPS = 1e-5
TILE_V = 2048

_NUM_CORES = 2
_NUM_SUBCORES = 16
_NUM_WORKERS = _NUM_CORES * _NUM_SUBCORES
_B_PER_W = BATCH // _NUM_WORKERS


_W_WORDS = _B_PER_W * EMBED_D


def _sc_gather_body(tflat_hbm, gidx_hbm, out_hbm, gidx_v, vals_v, sem):
    wid = lax.axis_index("s") * _NUM_CORES + lax.axis_index("c")
    pltpu.sync_copy(gidx_hbm.at[pl.ds(wid * _W_WORDS, _W_WORDS)], gidx_v)
    pltpu.async_copy(tflat_hbm.at[gidx_v], vals_v, sem).wait()
    pltpu.sync_copy(vals_v, out_hbm.at[pl.ds(wid * _W_WORDS, _W_WORDS)])


def _sc_gather(tflat, gidx):
    mesh = plsc.VectorSubcoreMesh(core_axis_name="c", subcore_axis_name="s")
    fn = pl.kernel(
        _sc_gather_body,
        mesh=mesh,
        out_type=jax.ShapeDtypeStruct((BATCH * EMBED_D,), jnp.float32),
        scratch_types=[
            pltpu.VMEM((_W_WORDS,), jnp.int32),
            pltpu.VMEM((_W_WORDS,), jnp.float32),
            pltpu.SemaphoreType.DMA,
        ],
        compiler_params=pltpu.CompilerParams(use_tc_tiling_on_sc=False),
    )
    return fn(tflat, gidx)


def _head_body(h0_ref, w1_ref, gammac_ref, betac_ref, whT_ref, bh_ref,
               out_ref, hT_ref):
    @pl.when(pl.program_id(0) == 0)
    def _():
        hT = lax.dot_general(w1_ref[...].astype(jnp.bfloat16),
                             h0_ref[...].astype(jnp.bfloat16),
                             (((1,), (1,)), ((), ())),
                             preferred_element_type=jnp.float32)
        hT = jnp.maximum(hT, 0.0)
        mean = jnp.mean(hT, axis=0, keepdims=True)
        cen = hT - mean
        var = jnp.mean(cen * cen, axis=0, keepdims=True)
        hTn = cen * lax.rsqrt(var + LN_EPS)
        hT_ref[...] = (hTn * gammac_ref[...] + betac_ref[...]).astype(
            jnp.bfloat16)

    acc = lax.dot_general(whT_ref[...].astype(jnp.bfloat16), hT_ref[...],
                          (((0,), (0,)), ((), ())),
                          preferred_element_type=jnp.float32)
    ones = jnp.ones((1, BATCH), dtype=jnp.bfloat16)
    bias = lax.dot_general(bh_ref[...].astype(jnp.bfloat16), ones,
                           (((0,), (0,)), ((), ())),
                           preferred_element_type=jnp.float32)
    out_ref[...] = acc + bias


def _head(h0, W1, gammac, betac, whT, bh2):
    grid = (pl.cdiv(VOCAB_SIZE, TILE_V),)
    return pl.pallas_call(
        _head_body,
        grid=grid,
        in_specs=[
            pl.BlockSpec((BATCH, EMBED_D), lambda i: (0, 0)),
            pl.BlockSpec((EMBED_D, EMBED_D), lambda i: (0, 0)),
            pl.BlockSpec((EMBED_D, 1), lambda i: (0, 0)),
            pl.BlockSpec((EMBED_D, 1), lambda i: (0, 0)),
            pl.BlockSpec((EMBED_D, TILE_V), lambda i: (0, i)),
            pl.BlockSpec((1, TILE_V), lambda i: (0, i)),
        ],
        out_specs=pl.BlockSpec((TILE_V, BATCH), lambda i: (i, 0)),
        out_shape=jax.ShapeDtypeStruct((VOCAB_SIZE, BATCH), jnp.float32),
        scratch_shapes=[pltpu.VMEM((EMBED_D, BATCH), jnp.bfloat16)],
    )(h0, W1, gammac, betac, whT, bh2)


def kernel(x, table, W1, gamma, beta, Wh, bh):
    tflat = table.T.reshape(VOCAB_SIZE * EMBED_D)
    gidx = (x.astype(jnp.int32)[:, None]
            + jnp.arange(EMBED_D, dtype=jnp.int32)[None, :] * VOCAB_SIZE
            ).reshape(BATCH * EMBED_D)
    h0 = _sc_gather(tflat, gidx).reshape(BATCH, EMBED_D)
    lt = _head(h0, W1, gamma.reshape(EMBED_D, 1), beta.reshape(EMBED_D, 1),
               Wh.T, bh.reshape(1, VOCAB_SIZE))
    return lt.T
